# baseline (speedup 1.0000x reference)
import jax
import jax.numpy as jnp
from jax import lax
from jax.experimental import pallas as pl
from jax.experimental.pallas import tpu as pltpu
from jax.experimental.pallas import tpu_sc as plsc

D = 1024
F = 286
FP = 288
NREL = 2
N = 10000
E = 64000
NR = NREL * N
NRP = 20480

BLK = 1000
NBLK = N // BLK

FC = 48
NPASS = FP // FC
PPC = NPASS // 2

NCORE = 2
NSUB = 16
EPT = E // NSUB
CH = 80
NCH = EPT // CH
RPT = NRP // NSUB
ZBR = 128


def _stage_a_body(h_ref, w_ref, b_ref, lnw_ref, *out_refs):
    zc_refs = out_refs[:NPASS]
    out0_ref = out_refs[NPASS]
    h = h_ref[...]
    var = jnp.mean(h * h, axis=1, keepdims=True)
    nx = lnw_ref[...] * (h * lax.rsqrt(var + 1e-6))
    p = lax.dot_general(nx, w_ref[...], (((1,), (0,)), ((), ())),
                        preferred_element_type=jnp.float32)
    lane = lax.broadcasted_iota(jnp.int32, (BLK, FC), 1)
    for q in range(NPASS):
        for r in range(NREL):
            blk = p[:, r * FP + q * FC:r * FP + (q + 1) * FC]
            if q == NPASS - 1:
                blk = jnp.where(lane == FC - 1, 1.0, blk)
            zc_refs[q][r] = blk
    out0_ref[...] = p[:, NREL * FP:NREL * FP + FP] + b_ref[...]


def _stage_a(h, wall, bias_p, lnw):
    zc_spec = pl.BlockSpec((NREL, BLK, FC), lambda i: (0, i, 0))
    return pl.pallas_call(
        _stage_a_body,
        grid=(NBLK,),
        in_specs=[
            pl.BlockSpec((BLK, D), lambda i: (i, 0)),
            pl.BlockSpec((D, 3 * FP), lambda i: (0, 0)),
            pl.BlockSpec((1, FP), lambda i: (0, 0)),
            pl.BlockSpec((1, D), lambda i: (0, 0)),
        ],
        out_specs=[zc_spec] * NPASS + [
            pl.BlockSpec((BLK, FP), lambda i: (i, 0)),
        ],
        out_shape=[jax.ShapeDtypeStruct((NREL, N, FC), jnp.float32)] * NPASS
        + [jax.ShapeDtypeStruct((N, FP), jnp.float32)],
    )(h, wall, bias_p, lnw)


def _sc_body(*refs):
    zc_hs = refs[:PPC]
    src_h, dst_h, et_h = refs[PPC:PPC + 3]
    acc_hs = refs[PPC + 3:2 * PPC + 3]
    (src_v, dst_v, et_v, gidx_v, sidx_v, rows_v, zb_v, acc_sh,
     sem0, sem1) = refs[2 * PPC + 3:]

    s = lax.axis_index("s")
    base = s * EPT

    pltpu.sync_copy(src_h.at[pl.ds(base, EPT)], src_v)
    pltpu.sync_copy(dst_h.at[pl.ds(base, EPT)], dst_v)
    pltpu.sync_copy(et_h.at[pl.ds(base, EPT)], et_v)

    def _zzb(i, _):
        for b in range(FC // 16):
            zb_v[i, pl.ds(b * 16, 16)] = jnp.zeros((16,), jnp.float32)
        return 0
    lax.fori_loop(0, ZBR, _zzb, 0)

    for j in range(EPT // 16):
        sv = src_v[pl.ds(j * 16, 16)]
        dv = dst_v[pl.ds(j * 16, 16)]
        tv = et_v[pl.ds(j * 16, 16)]
        gidx_v[j // (CH // 16), pl.ds((j % (CH // 16)) * 16, 16)] = tv * N + sv
        sidx_v[j // (CH // 16), pl.ds((j % (CH // 16)) * 16, 16)] = tv * N + dv

    def run_pass(zc_h, acc_h):
        for q in range(RPT // ZBR):
            pltpu.sync_copy(zb_v, acc_sh.at[pl.ds(s * RPT + q * ZBR, ZBR)])
        plsc.subcore_barrier()

        sems = (sem0, sem1)
        cps = [None, None]
        cps[0] = pltpu.async_copy(zc_h.at[gidx_v.at[0]], rows_v.at[0],
                                  sems[0])
        for j in range(NCH):
            if j + 1 < NCH:
                cps[(j + 1) % 2] = pltpu.async_copy(
                    zc_h.at[gidx_v.at[j + 1]], rows_v.at[(j + 1) % 2],
                    sems[(j + 1) % 2])
            cps[j % 2].wait()
            pltpu.sync_copy(rows_v.at[j % 2], acc_sh.at[sidx_v.at[j]],
                            add=True)
        plsc.subcore_barrier()

        pltpu.sync_copy(acc_sh.at[pl.ds(s * RPT, RPT)],
                        acc_h.at[pl.ds(s * RPT, RPT)])
        plsc.subcore_barrier()

    for q in range(PPC):
        run_pass(zc_hs[q], acc_hs[q])


def _sc_stage(zcs, src, dst, et):
    mesh = plsc.VectorSubcoreMesh(core_axis_name="c", subcore_axis_name="s",
                                  num_cores=1)
    f32 = jnp.float32
    kern = pl.kernel(
        _sc_body,
        mesh=mesh,
        compiler_params=pltpu.CompilerParams(use_tc_tiling_on_sc=False),
        out_type=[jax.ShapeDtypeStruct((NRP, FC), f32)] * PPC,
        scratch_types=[
            pltpu.VMEM((EPT,), jnp.int32),
            pltpu.VMEM((EPT,), jnp.int32),
            pltpu.VMEM((EPT,), jnp.int32),
            pltpu.VMEM((NCH, CH), jnp.int32),
            pltpu.VMEM((NCH, CH), jnp.int32),
            pltpu.VMEM((2, CH, FC), f32),
            pltpu.VMEM((ZBR, FC), f32),
            pltpu.VMEM_SHARED((NRP, FC), f32),
            pltpu.SemaphoreType.DMA,
            pltpu.SemaphoreType.DMA,
        ],
    )
    return (list(kern(*zcs[:PPC], src, dst, et)) +
            list(kern(*zcs[PPC:], src, dst, et)))


def _stage_c_body(*refs):
    out0_ref = refs[0]
    a_refs = refs[1:1 + NPASS]
    h_ref, wo_ref, res_ref, s_ref = refs[1 + NPASS:]
    r = pl.program_id(1)
    asum = [a[...] for a in a_refs]
    agg = jnp.concatenate(asum, axis=1)
    d = asum[NPASS - 1][:, FC - 1:FC]
    term = agg * (1.0 / jnp.maximum(d, 1.0))

    @pl.when(r == 0)
    def _():
        s_ref[...] = out0_ref[...] + term

    @pl.when(r == 1)
    def _():
        v = s_ref[...] + term
        y = jnp.where(v > 0, v, jnp.exp(v) - 1.0)
        res_ref[...] = h_ref[...] + lax.dot_general(
            y, wo_ref[...], (((1,), (0,)), ((), ())),
            preferred_element_type=jnp.float32)


def _stage_c(out0, accs, h, wo_p):
    acc_spec = pl.BlockSpec((BLK, FC), lambda i, r: (r * NBLK + i, 0))
    return pl.pallas_call(
        _stage_c_body,
        grid=(NBLK, NREL),
        in_specs=[pl.BlockSpec((BLK, FP), lambda i, r: (i, 0))] +
                 [acc_spec] * NPASS + [
            pl.BlockSpec((BLK, D), lambda i, r: (i, 0)),
            pl.BlockSpec((FP, D), lambda i, r: (0, 0)),
        ],
        out_specs=pl.BlockSpec((BLK, D), lambda i, r: (i, 0)),
        out_shape=jax.ShapeDtypeStruct((N, D), jnp.float32),
        scratch_shapes=[pltpu.VMEM((BLK, FP), jnp.float32)],
    )(out0, *accs, h, wo_p)


def kernel(hidden_states, ln_weight, W_rel, W_root, conv_bias, Wo,
           edge_indices, edge_type):
    src = edge_indices[0].astype(jnp.int32)
    dst = edge_indices[1].astype(jnp.int32)
    et = edge_type.astype(jnp.int32)

    wrel_p = jnp.pad(W_rel, ((0, 0), (0, 0), (0, FP - F)))
    wroot_p = jnp.pad(W_root, ((0, 0), (0, FP - F)))
    wall = jnp.concatenate([wrel_p[0], wrel_p[1], wroot_p], axis=1)
    bias_p = jnp.pad(conv_bias, (0, FP - F)).reshape(1, FP)
    wo_p = jnp.pad(Wo, ((0, FP - F), (0, 0)))
    lnw = ln_weight.reshape(1, D)

    *zcs, out0 = _stage_a(hidden_states, wall, bias_p, lnw)
    zcs = [z.reshape(NR, FC) for z in zcs]
    accs = _sc_stage(zcs, src, dst, et)
    return _stage_c(out0, accs, hidden_states, wo_p)


# speedup vs baseline: 1.1736x; 1.1736x over previous
import jax
import jax.numpy as jnp
from jax import lax
from jax.experimental import pallas as pl
from jax.experimental.pallas import tpu as pltpu
from jax.experimental.pallas import tpu_sc as plsc

D = 1024
F = 286
FP = 288
NREL = 2
N = 10000
E = 64000
NR = NREL * N
NRP = 20480

BLK = 1000
NBLK = N // BLK

FC = 48
NPASS = FP // FC
PPC = NPASS // 2

NCORE = 2
NSUB = 16
EPT = E // NSUB
CH = 80
NCH = EPT // CH
RPT = NRP // NSUB
ZBR = 128


def _stage_a_body(h_ref, w_ref, b_ref, lnw_ref, *out_refs):
    zc_refs = out_refs[:NPASS]
    out0_ref = out_refs[NPASS]
    h = h_ref[...]
    var = jnp.mean(h * h, axis=1, keepdims=True)
    nx = lnw_ref[...] * (h * lax.rsqrt(var + 1e-6))
    p = lax.dot_general(nx, w_ref[...], (((1,), (0,)), ((), ())),
                        preferred_element_type=jnp.float32)
    lane = lax.broadcasted_iota(jnp.int32, (BLK, FC), 1)
    for q in range(NPASS):
        for r in range(NREL):
            blk = p[:, r * FP + q * FC:r * FP + (q + 1) * FC]
            if q == NPASS - 1:
                blk = jnp.where(lane == FC - 1, 1.0, blk)
            zc_refs[q][r] = blk
    out0_ref[...] = p[:, NREL * FP:NREL * FP + FP] + b_ref[...]


def _stage_a(h, wall, bias_p, lnw):
    zc_spec = pl.BlockSpec((NREL, BLK, FC), lambda i: (0, i, 0))
    return pl.pallas_call(
        _stage_a_body,
        grid=(NBLK,),
        in_specs=[
            pl.BlockSpec((BLK, D), lambda i: (i, 0)),
            pl.BlockSpec((D, 3 * FP), lambda i: (0, 0)),
            pl.BlockSpec((1, FP), lambda i: (0, 0)),
            pl.BlockSpec((1, D), lambda i: (0, 0)),
        ],
        out_specs=[zc_spec] * NPASS + [
            pl.BlockSpec((BLK, FP), lambda i: (i, 0)),
        ],
        out_shape=[jax.ShapeDtypeStruct((NREL, N, FC), jnp.float32)] * NPASS
        + [jax.ShapeDtypeStruct((N, FP), jnp.float32)],
    )(h, wall, bias_p, lnw)


def _sc_body(*refs):
    zc_hs = refs[:NPASS]
    src_h, dst_h, et_h = refs[NPASS:NPASS + 3]
    acc_hs = refs[NPASS + 3:2 * NPASS + 3]
    (src_v, dst_v, et_v, gidx_v, sidx_v, rows_v, zb_v, acc_sh,
     sem0, sem1) = refs[2 * NPASS + 3:]

    c = lax.axis_index("c")
    s = lax.axis_index("s")
    base = s * EPT

    pltpu.sync_copy(src_h.at[pl.ds(base, EPT)], src_v)
    pltpu.sync_copy(dst_h.at[pl.ds(base, EPT)], dst_v)
    pltpu.sync_copy(et_h.at[pl.ds(base, EPT)], et_v)

    def _zzb(i, _):
        for b in range(FC // 16):
            zb_v[i, pl.ds(b * 16, 16)] = jnp.zeros((16,), jnp.float32)
        return 0
    lax.fori_loop(0, ZBR, _zzb, 0)

    for j in range(EPT // 16):
        sv = src_v[pl.ds(j * 16, 16)]
        dv = dst_v[pl.ds(j * 16, 16)]
        tv = et_v[pl.ds(j * 16, 16)]
        gidx_v[j // (CH // 16), pl.ds((j % (CH // 16)) * 16, 16)] = tv * N + sv
        sidx_v[j // (CH // 16), pl.ds((j % (CH // 16)) * 16, 16)] = tv * N + dv

    def run_pass(zc_h, acc_h):
        for q in range(RPT // ZBR):
            pltpu.sync_copy(zb_v, acc_sh.at[pl.ds(s * RPT + q * ZBR, ZBR)])
        plsc.subcore_barrier()

        sems = (sem0, sem1)
        cps = [None, None]
        cps[0] = pltpu.async_copy(zc_h.at[gidx_v.at[0]], rows_v.at[0],
                                  sems[0])
        for j in range(NCH):
            if j + 1 < NCH:
                cps[(j + 1) % 2] = pltpu.async_copy(
                    zc_h.at[gidx_v.at[j + 1]], rows_v.at[(j + 1) % 2],
                    sems[(j + 1) % 2])
            cps[j % 2].wait()
            pltpu.sync_copy(rows_v.at[j % 2], acc_sh.at[sidx_v.at[j]],
                            add=True)
        plsc.subcore_barrier()

        pltpu.sync_copy(acc_sh.at[pl.ds(s * RPT, RPT)],
                        acc_h.at[pl.ds(s * RPT, RPT)])
        plsc.subcore_barrier()

    for cc in range(NCORE):
        @pl.when(c == cc)
        def _():
            for pp in range(PPC):
                q = cc * PPC + pp
                run_pass(zc_hs[q], acc_hs[q])


def _sc_stage(zcs, src, dst, et):
    mesh = plsc.VectorSubcoreMesh(core_axis_name="c", subcore_axis_name="s")
    f32 = jnp.float32
    kern = pl.kernel(
        _sc_body,
        mesh=mesh,
        compiler_params=pltpu.CompilerParams(use_tc_tiling_on_sc=False),
        out_type=[jax.ShapeDtypeStruct((NRP, FC), f32)] * NPASS,
        scratch_types=[
            pltpu.VMEM((EPT,), jnp.int32),
            pltpu.VMEM((EPT,), jnp.int32),
            pltpu.VMEM((EPT,), jnp.int32),
            pltpu.VMEM((NCH, CH), jnp.int32),
            pltpu.VMEM((NCH, CH), jnp.int32),
            pltpu.VMEM((2, CH, FC), f32),
            pltpu.VMEM((ZBR, FC), f32),
            pltpu.VMEM_SHARED((NRP, FC), f32),
            pltpu.SemaphoreType.DMA,
            pltpu.SemaphoreType.DMA,
        ],
    )
    return kern(*zcs, src, dst, et)


def _stage_c_body(*refs):
    out0_ref = refs[0]
    a_refs = refs[1:1 + NPASS]
    h_ref, wo_ref, res_ref, s_ref = refs[1 + NPASS:]
    r = pl.program_id(1)
    asum = [a[...] for a in a_refs]
    agg = jnp.concatenate(asum, axis=1)
    d = asum[NPASS - 1][:, FC - 1:FC]
    term = agg * (1.0 / jnp.maximum(d, 1.0))

    @pl.when(r == 0)
    def _():
        s_ref[...] = out0_ref[...] + term

    @pl.when(r == 1)
    def _():
        v = s_ref[...] + term
        y = jnp.where(v > 0, v, jnp.exp(v) - 1.0)
        res_ref[...] = h_ref[...] + lax.dot_general(
            y, wo_ref[...], (((1,), (0,)), ((), ())),
            preferred_element_type=jnp.float32)


def _stage_c(out0, accs, h, wo_p):
    acc_spec = pl.BlockSpec((BLK, FC), lambda i, r: (r * NBLK + i, 0))
    return pl.pallas_call(
        _stage_c_body,
        grid=(NBLK, NREL),
        in_specs=[pl.BlockSpec((BLK, FP), lambda i, r: (i, 0))] +
                 [acc_spec] * NPASS + [
            pl.BlockSpec((BLK, D), lambda i, r: (i, 0)),
            pl.BlockSpec((FP, D), lambda i, r: (0, 0)),
        ],
        out_specs=pl.BlockSpec((BLK, D), lambda i, r: (i, 0)),
        out_shape=jax.ShapeDtypeStruct((N, D), jnp.float32),
        scratch_shapes=[pltpu.VMEM((BLK, FP), jnp.float32)],
    )(out0, *accs, h, wo_p)


def kernel(hidden_states, ln_weight, W_rel, W_root, conv_bias, Wo,
           edge_indices, edge_type):
    src = edge_indices[0].astype(jnp.int32)
    dst = edge_indices[1].astype(jnp.int32)
    et = edge_type.astype(jnp.int32)

    wrel_p = jnp.pad(W_rel, ((0, 0), (0, 0), (0, FP - F)))
    wroot_p = jnp.pad(W_root, ((0, 0), (0, FP - F)))
    wall = jnp.concatenate([wrel_p[0], wrel_p[1], wroot_p], axis=1)
    bias_p = jnp.pad(conv_bias, (0, FP - F)).reshape(1, FP)
    wo_p = jnp.pad(Wo, ((0, FP - F), (0, 0)))
    lnw = ln_weight.reshape(1, D)

    *zcs, out0 = _stage_a(hidden_states, wall, bias_p, lnw)
    zcs = [z.reshape(NR, FC) for z in zcs]
    accs = _sc_stage(zcs, src, dst, et)
    return _stage_c(out0, accs, hidden_states, wo_p)
